# baseline (speedup 1.0000x reference)
import functools

import jax
import jax.numpy as jnp
import numpy as np
from jax import lax
from jax.experimental import pallas as pl
from jax.experimental.pallas import tpu as pltpu
from jax.experimental.pallas import tpu_sc as plsc

BS = 128
N = 2048
C = 32
Q = 64
K = 5
F = N // 2 + 1
FP = 1040
CPS = 2


def _dft_tables():
    t = np.arange(N)[None, :]
    f = np.arange(F)[:, None]
    ang = 2.0 * np.pi * f * t / N
    cs = np.zeros((2 * FP, N), dtype=np.float32)
    cs[:F, :] = np.cos(ang)
    cs[FP:FP + F, :] = -np.sin(ang)
    return jnp.asarray(cs)


def _split(v):
    hi = v.astype(jnp.bfloat16)
    lo = (v - hi.astype(jnp.float32)).astype(jnp.bfloat16)
    return hi, lo


_MN = (((1,), (0,)), ((), ()))


def _dot3(ah, al, bh, bl, dn):
    d = functools.partial(
        jax.lax.dot_general, dimension_numbers=dn,
        preferred_element_type=jnp.float32)
    return d(ah, bh) + (d(ah, bl) + d(al, bh))


def _spectrum_kernel(cs_ref, xt_ref, wt_ref, t2_ref, gidx_ref, coef_ref):
    xspec = jax.lax.dot_general(
        cs_ref[...], xt_ref[...], _MN,
        precision=jax.lax.Precision.HIGHEST,
        preferred_element_type=jnp.float32)
    wr = jax.lax.dot_general(
        cs_ref[...], wt_ref[...], _MN,
        precision=jax.lax.Precision.HIGHEST,
        preferred_element_type=jnp.float32)

    xr = xspec[:FP, :]
    xi = xspec[FP:, :]
    mag2 = xr * xr + xi * xi

    iota_f = jax.lax.broadcasted_iota(jnp.int32, (FP, CPS * BS), 0)
    alpha = jnp.where(
        jnp.logical_or(iota_f == 0, iota_f == N // 2),
        jnp.float32(1.0 / N), jnp.float32(2.0 / N))
    xra = xr * alpha
    xia = xi * alpha

    idx_rows = []
    cr_rows = []
    ci_rows = []
    for _ in range(K):
        m = jnp.max(mag2, axis=0, keepdims=True)
        amax = jnp.min(jnp.where(mag2 == m, iota_f, jnp.int32(2**30)),
                       axis=0, keepdims=True)
        pick = iota_f == amax
        idx_rows.append(amax)
        cr_rows.append(jnp.sum(jnp.where(pick, xra, 0.0), axis=0,
                               keepdims=True))
        ci_rows.append(jnp.sum(jnp.where(pick, xia, 0.0), axis=0,
                               keepdims=True))
        mag2 = jnp.where(pick, -1.0, mag2)

    zf = jnp.zeros((1, CPS * BS), jnp.float32)
    coef16 = jnp.concatenate(
        cr_rows + [zf, zf, zf] + ci_rows + [zf, zf, zf], axis=0)

    step = pl.program_id(0)
    for j in range(CPS):
        cols = slice(j * BS, (j + 1) * BS)
        qcols = slice(j * Q, (j + 1) * Q)
        t2_ref[j] = jnp.concatenate(
            [wr[:FP, qcols], wr[FP:, qcols]], axis=1)
        gidx_ref[j] = (jnp.concatenate([r[:, cols] for r in idx_rows],
                                       axis=1)
                       + (step * CPS + j) * FP)
        coef_ref[j] = coef16[:, cols]


def _sc_gather(t2, gidx):
    mesh = plsc.VectorSubcoreMesh(core_axis_name="c", subcore_axis_name="s")

    @functools.partial(
        pl.kernel, mesh=mesh,
        out_type=jax.ShapeDtypeStruct((C, K * BS, 128), jnp.float32),
        scratch_types=[
            pltpu.VMEM((K * BS,), jnp.int32),
            pltpu.VMEM((K * BS, 128), jnp.float32),
            pltpu.SemaphoreType.DMA,
        ],
    )
    def body(t2_hbm, gidx_hbm, out_hbm, fidx_v, rows_v, sem):
        wid = lax.axis_index("s") * 2 + lax.axis_index("c")
        pltpu.sync_copy(gidx_hbm.at[wid], fidx_v)
        pltpu.async_copy(t2_hbm.at[fidx_v], rows_v, sem).wait()
        pltpu.sync_copy(rows_v, out_hbm.at[wid])

    return body(t2, gidx)


def _combine_kernel(rows_ref, coef_ref, b_ref, g_ref, out_ref):
    c = pl.program_id(0)
    rows3 = rows_ref[0].reshape(K, BS, 128)
    cr5 = coef_ref[0][:K, :]
    ci5 = coef_ref[0][8:8 + K, :]
    m3 = jnp.concatenate(
        [jnp.broadcast_to(cr5[:, :, None], (K, BS, Q)),
         jnp.broadcast_to(ci5[:, :, None], (K, BS, Q))], axis=2)
    contrib = jnp.sum(rows3 * m3, axis=0)
    acc = contrib[:, :Q] + contrib[:, Q:]

    @pl.when(c == 0)
    def _init():
        out_ref[...] = acc

    @pl.when(c > 0)
    def _accum():
        out_ref[...] += acc

    @pl.when(c == C - 1)
    def _epilogue():
        z = out_ref[...] + b_ref[...] + g_ref[...]
        m = jnp.max(z, axis=1, keepdims=True)
        e = jnp.exp(z - m)
        y = e / jnp.sum(e, axis=1, keepdims=True)
        iota_q = jax.lax.broadcasted_iota(jnp.int32, (BS, Q), 1)
        first = jnp.min(jnp.where(z == m, iota_q, jnp.int32(2**30)),
                        axis=1, keepdims=True)
        hard = jnp.where(iota_q == first, jnp.float32(1.0), jnp.float32(0.0))
        out_ref[...] = (hard - y) + y


@jax.jit
def _run(x, W, b, cs, g):
    xt = jnp.transpose(x, (1, 2, 0)).reshape(N, C * BS)
    wt = jnp.transpose(W.reshape(Q, N, C), (1, 2, 0)).reshape(N, C * Q)
    bb = b.reshape(1, Q)

    t2, gidx, coef = pl.pallas_call(
        _spectrum_kernel,
        grid=(C // CPS,),
        in_specs=[
            pl.BlockSpec((2 * FP, N), lambda i: (0, 0)),
            pl.BlockSpec((N, CPS * BS), lambda i: (0, i)),
            pl.BlockSpec((N, CPS * Q), lambda i: (0, i)),
        ],
        out_specs=[
            pl.BlockSpec((CPS, FP, 128), lambda i: (i, 0, 0)),
            pl.BlockSpec((CPS, 1, K * BS), lambda i: (i, 0, 0)),
            pl.BlockSpec((CPS, 16, BS), lambda i: (i, 0, 0)),
        ],
        out_shape=[
            jax.ShapeDtypeStruct((C, FP, 128), jnp.float32),
            jax.ShapeDtypeStruct((C, 1, K * BS), jnp.int32),
            jax.ShapeDtypeStruct((C, 16, BS), jnp.float32),
        ],
    )(cs, xt, wt)

    rows = _sc_gather(t2.reshape(C * FP, 128), gidx.reshape(C, K * BS))

    return pl.pallas_call(
        _combine_kernel,
        grid=(C,),
        in_specs=[
            pl.BlockSpec((1, K * BS, 128), lambda i: (i, 0, 0)),
            pl.BlockSpec((1, 16, BS), lambda i: (i, 0, 0)),
            pl.BlockSpec((1, Q), lambda i: (0, 0)),
            pl.BlockSpec((BS, Q), lambda i: (0, 0)),
        ],
        out_specs=pl.BlockSpec((BS, Q), lambda i: (0, 0)),
        out_shape=jax.ShapeDtypeStruct((BS, Q), jnp.float32),
    )(rows, coef, bb, g)


def kernel(x, W, b):
    cs = _dft_tables()
    g = jax.random.gumbel(jax.random.key(42), (BS, Q), dtype=jnp.float32)
    return _run(x, W, b, cs, g)


# speedup vs baseline: 1.0001x; 1.0001x over previous
import functools

import jax
import jax.numpy as jnp
import numpy as np
from jax import lax
from jax.experimental import pallas as pl
from jax.experimental.pallas import tpu as pltpu
from jax.experimental.pallas import tpu_sc as plsc

BS = 128
N = 2048
C = 32
Q = 64
K = 5
F = N // 2 + 1
FP = 1040
CPS = 2


def _dft_tables():
    t = np.arange(N)[None, :]
    f = np.arange(F)[:, None]
    ang = 2.0 * np.pi * f * t / N
    cs = np.zeros((2 * FP, N), dtype=np.float32)
    cs[:F, :] = np.cos(ang)
    cs[FP:FP + F, :] = -np.sin(ang)
    return jnp.asarray(cs)


_MN = (((1,), (0,)), ((), ()))


def _spectrum_kernel(cs_ref, xt_ref, wt_ref, t2_ref, gidx_ref, coef_ref):
    xspec = jax.lax.dot_general(
        cs_ref[...], xt_ref[...], _MN,
        precision=jax.lax.Precision.HIGHEST,
        preferred_element_type=jnp.float32)
    wr = jax.lax.dot_general(
        cs_ref[...], wt_ref[...], _MN,
        precision=jax.lax.Precision.HIGHEST,
        preferred_element_type=jnp.float32)

    xr = xspec[:FP, :]
    xi = xspec[FP:, :]
    mag2 = xr * xr + xi * xi

    iota_f = jax.lax.broadcasted_iota(jnp.int32, (FP, CPS * BS), 0)
    alpha = jnp.where(
        jnp.logical_or(iota_f == 0, iota_f == N // 2),
        jnp.float32(1.0 / N), jnp.float32(2.0 / N))
    xra = xr * alpha
    xia = xi * alpha

    idx_rows = []
    cr_rows = []
    ci_rows = []
    for _ in range(K):
        m = jnp.max(mag2, axis=0, keepdims=True)
        amax = jnp.min(jnp.where(mag2 == m, iota_f, jnp.int32(2**30)),
                       axis=0, keepdims=True)
        pick = iota_f == amax
        idx_rows.append(amax)
        cr_rows.append(jnp.sum(jnp.where(pick, xra, 0.0), axis=0,
                               keepdims=True))
        ci_rows.append(jnp.sum(jnp.where(pick, xia, 0.0), axis=0,
                               keepdims=True))
        mag2 = jnp.where(pick, -1.0, mag2)

    zf = jnp.zeros((1, CPS * BS), jnp.float32)
    coef16 = jnp.concatenate(
        cr_rows + [zf, zf, zf] + ci_rows + [zf, zf, zf], axis=0)

    step = pl.program_id(0)
    for j in range(CPS):
        cols = slice(j * BS, (j + 1) * BS)
        qcols = slice(j * Q, (j + 1) * Q)
        t2_ref[j] = jnp.concatenate(
            [wr[:FP, qcols], wr[FP:, qcols]], axis=1)
        gidx_ref[j] = (jnp.concatenate([r[:, cols] for r in idx_rows],
                                       axis=1)
                       + (step * CPS + j) * FP)
        coef_ref[j] = coef16[:, cols]


def _sc_gather(t2, gidx):
    mesh = plsc.VectorSubcoreMesh(core_axis_name="c", subcore_axis_name="s")

    @functools.partial(
        pl.kernel, mesh=mesh,
        out_type=jax.ShapeDtypeStruct((C, K * BS, 128), jnp.float32),
        scratch_types=[
            pltpu.VMEM((K * BS,), jnp.int32),
            pltpu.VMEM((K * BS, 128), jnp.float32),
            pltpu.SemaphoreType.DMA,
        ],
    )
    def body(t2_hbm, gidx_hbm, out_hbm, fidx_v, rows_v, sem):
        wid = lax.axis_index("s") * 2 + lax.axis_index("c")
        pltpu.sync_copy(gidx_hbm.at[wid], fidx_v)
        pltpu.async_copy(t2_hbm.at[fidx_v], rows_v, sem).wait()
        pltpu.sync_copy(rows_v, out_hbm.at[wid])

    return body(t2, gidx)


def _combine_kernel(rows_ref, coef_ref, b_ref, g_ref, out_ref):
    c = pl.program_id(0)
    rows3 = rows_ref[0].reshape(K, BS, 128)
    cr5 = coef_ref[0][:K, :]
    ci5 = coef_ref[0][8:8 + K, :]
    m3 = jnp.concatenate(
        [jnp.broadcast_to(cr5[:, :, None], (K, BS, Q)),
         jnp.broadcast_to(ci5[:, :, None], (K, BS, Q))], axis=2)
    contrib = jnp.sum(rows3 * m3, axis=0)
    acc = contrib[:, :Q] + contrib[:, Q:]

    @pl.when(c == 0)
    def _init():
        out_ref[...] = acc

    @pl.when(c > 0)
    def _accum():
        out_ref[...] += acc

    @pl.when(c == C - 1)
    def _epilogue():
        z = out_ref[...] + b_ref[...] + g_ref[...]
        m = jnp.max(z, axis=1, keepdims=True)
        e = jnp.exp(z - m)
        y = e / jnp.sum(e, axis=1, keepdims=True)
        iota_q = jax.lax.broadcasted_iota(jnp.int32, (BS, Q), 1)
        first = jnp.min(jnp.where(z == m, iota_q, jnp.int32(2**30)),
                        axis=1, keepdims=True)
        hard = jnp.where(iota_q == first, jnp.float32(1.0), jnp.float32(0.0))
        out_ref[...] = (hard - y) + y


@jax.jit
def _run(x, W, b, cs, g):
    xt = jnp.transpose(x, (1, 2, 0)).reshape(N, C * BS)
    wt = jnp.transpose(W.reshape(Q, N, C), (1, 2, 0)).reshape(N, C * Q)
    bb = b.reshape(1, Q)

    t2, gidx, coef = pl.pallas_call(
        _spectrum_kernel,
        grid=(C // CPS,),
        in_specs=[
            pl.BlockSpec((2 * FP, N), lambda i: (0, 0)),
            pl.BlockSpec((N, CPS * BS), lambda i: (0, i)),
            pl.BlockSpec((N, CPS * Q), lambda i: (0, i)),
        ],
        out_specs=[
            pl.BlockSpec((CPS, FP, 128), lambda i: (i, 0, 0)),
            pl.BlockSpec((CPS, 1, K * BS), lambda i: (i, 0, 0)),
            pl.BlockSpec((CPS, 16, BS), lambda i: (i, 0, 0)),
        ],
        out_shape=[
            jax.ShapeDtypeStruct((C, FP, 128), jnp.float32),
            jax.ShapeDtypeStruct((C, 1, K * BS), jnp.int32),
            jax.ShapeDtypeStruct((C, 16, BS), jnp.float32),
        ],
    )(cs, xt, wt)

    rows = _sc_gather(t2.reshape(C * FP, 128), gidx.reshape(C, K * BS))

    return pl.pallas_call(
        _combine_kernel,
        grid=(C,),
        in_specs=[
            pl.BlockSpec((1, K * BS, 128), lambda i: (i, 0, 0)),
            pl.BlockSpec((1, 16, BS), lambda i: (i, 0, 0)),
            pl.BlockSpec((1, Q), lambda i: (0, 0)),
            pl.BlockSpec((BS, Q), lambda i: (0, 0)),
        ],
        out_specs=pl.BlockSpec((BS, Q), lambda i: (0, 0)),
        out_shape=jax.ShapeDtypeStruct((BS, Q), jnp.float32),
    )(rows, coef, bb, g)


def kernel(x, W, b):
    cs = _dft_tables()
    g = jax.random.gumbel(jax.random.key(42), (BS, Q), dtype=jnp.float32)
    return _run(x, W, b, cs, g)
